# baseline (speedup 1.0000x reference)
import functools

import jax
import jax.numpy as jnp
from jax import lax
from jax.experimental import pallas as pl
from jax.experimental.pallas import tpu as pltpu
from jax.experimental.pallas import tpu_sc as plsc

VOCAB = 50257
EMB = 1024
SEQ = 2048

_NC = 2
_NS = 16
_NW = _NC * _NS
_B_PER_W = SEQ // _NW


@functools.cache
def _make_sc_gather():
    @functools.partial(
        pl.kernel,
        mesh=plsc.VectorSubcoreMesh(
            core_axis_name="c", subcore_axis_name="s", num_cores=_NC
        ),
        out_type=jax.ShapeDtypeStruct((SEQ, EMB), jnp.float32),
        scratch_types=[
            pltpu.VMEM((_B_PER_W,), jnp.int32),
            pltpu.VMEM((_B_PER_W, EMB), jnp.float32),
            pltpu.SemaphoreType.DMA,
        ],
    )
    def _sc_gather(idx_hbm, table_hbm, out_hbm, idx_v, rows_v, sem):
        wid = lax.axis_index("s") * _NC + lax.axis_index("c")
        base = wid * _B_PER_W
        pltpu.sync_copy(idx_hbm.at[pl.ds(base, _B_PER_W)], idx_v)
        pltpu.async_copy(table_hbm.at[idx_v], rows_v, sem).wait()
        pltpu.sync_copy(rows_v, out_hbm.at[pl.ds(base, _B_PER_W)])

    return _sc_gather


_TV = 512
_NV = (VOCAB + _TV - 1) // _TV


def _tc_body(emb_ref, pos_ref, scale_ref, bias_ref, w_ref, out_ref, h_ref):
    @pl.when(pl.program_id(0) == 0)
    def _():
        h = emb_ref[...] + pos_ref[...]
        mean = jnp.mean(h, axis=1, keepdims=True)
        hc = h - mean
        var = jnp.mean(hc * hc, axis=1, keepdims=True)
        hn = hc * lax.rsqrt(var + 1e-5)
        h_ref[...] = (hn * scale_ref[...] + bias_ref[...]).astype(jnp.bfloat16)

    out_ref[0, ...] = lax.dot_general(
        h_ref[...],
        w_ref[...].astype(jnp.bfloat16),
        (((1,), (1,)), ((), ())),
        preferred_element_type=jnp.float32,
    )


_tc_head = pl.pallas_call(
    _tc_body,
    grid=(_NV,),
    in_specs=[
        pl.BlockSpec((SEQ, EMB), lambda i: (0, 0)),
        pl.BlockSpec((SEQ, EMB), lambda i: (0, 0)),
        pl.BlockSpec((1, EMB), lambda i: (0, 0)),
        pl.BlockSpec((1, EMB), lambda i: (0, 0)),
        pl.BlockSpec((_TV, EMB), lambda i: (i, 0)),
    ],
    out_specs=pl.BlockSpec((1, SEQ, _TV), lambda i: (0, 0, i)),
    out_shape=jax.ShapeDtypeStruct((1, SEQ, VOCAB), jnp.float32),
    scratch_shapes=[pltpu.VMEM((SEQ, EMB), jnp.bfloat16)],
    compiler_params=pltpu.CompilerParams(
        dimension_semantics=("arbitrary",),
    ),
)


def kernel(x, tok_table, pos_table, ln_scale, ln_bias, W_out):
    batch, seq = x.shape
    idx = x.reshape(seq).astype(jnp.int32)
    emb = _make_sc_gather()(idx, tok_table)
    return _tc_head(
        emb,
        pos_table[:seq],
        ln_scale.reshape(1, EMB),
        ln_bias.reshape(1, EMB),
        W_out,
    )


# speedup vs baseline: 1.3800x; 1.3800x over previous
import functools

import jax
import jax.numpy as jnp
from jax import lax
from jax.experimental import pallas as pl
from jax.experimental.pallas import tpu as pltpu
from jax.experimental.pallas import tpu_sc as plsc

VOCAB = 50257
EMB = 1024
SEQ = 2048

_NC = 2
_NS = 16
_NW = _NC * _NS
_B_PER_W = SEQ // _NW


@functools.cache
def _make_sc_gather():
    @functools.partial(
        pl.kernel,
        mesh=plsc.VectorSubcoreMesh(
            core_axis_name="c", subcore_axis_name="s", num_cores=_NC
        ),
        out_type=jax.ShapeDtypeStruct((SEQ, EMB), jnp.float32),
        scratch_types=[
            pltpu.VMEM((_B_PER_W,), jnp.int32),
            pltpu.VMEM((_B_PER_W, EMB), jnp.float32),
            pltpu.SemaphoreType.DMA,
        ],
    )
    def _sc_gather(idx_hbm, table_hbm, out_hbm, idx_v, rows_v, sem):
        wid = lax.axis_index("s") * _NC + lax.axis_index("c")
        base = wid * _B_PER_W
        pltpu.sync_copy(idx_hbm.at[pl.ds(base, _B_PER_W)], idx_v)
        pltpu.async_copy(table_hbm.at[idx_v], rows_v, sem).wait()
        pltpu.sync_copy(rows_v, out_hbm.at[pl.ds(base, _B_PER_W)])

    return _sc_gather


_TV = 512
_NV = (VOCAB + _TV - 1) // _TV


def _ln_body(emb_ref, pos_ref, scale_ref, bias_ref, h_ref):
    h = emb_ref[...] + pos_ref[...]
    mean = jnp.mean(h, axis=1, keepdims=True)
    hc = h - mean
    var = jnp.mean(hc * hc, axis=1, keepdims=True)
    hn = hc * lax.rsqrt(var + 1e-5)
    h_ref[...] = (hn * scale_ref[...] + bias_ref[...]).astype(jnp.bfloat16)


_tc_ln = pl.pallas_call(
    _ln_body,
    out_shape=jax.ShapeDtypeStruct((SEQ, EMB), jnp.bfloat16),
)


def _mm_body(h_ref, w_ref, out_ref):
    out_ref[...] = lax.dot_general(
        h_ref[...],
        w_ref[...].astype(jnp.bfloat16),
        (((1,), (1,)), ((), ())),
        preferred_element_type=jnp.float32,
    )


def _make_chunk(off_blocks, n_blocks, width):
    return pl.pallas_call(
        _mm_body,
        grid=(n_blocks,),
        in_specs=[
            pl.BlockSpec((SEQ, EMB), lambda i: (0, 0)),
            pl.BlockSpec((_TV, EMB), lambda i, o=off_blocks: (i + o, 0)),
        ],
        out_specs=pl.BlockSpec((SEQ, _TV), lambda i: (0, i)),
        out_shape=jax.ShapeDtypeStruct((SEQ, width), jnp.float32),
        compiler_params=pltpu.CompilerParams(
            dimension_semantics=("arbitrary",),
        ),
    )


_N_CHUNKS = 4
_chunk_calls = []
_off = 0
for _c in range(_N_CHUNKS):
    _nb = (_NV - _off + (_N_CHUNKS - 1 - _c)) // (_N_CHUNKS - _c)
    _width = min(_nb * _TV, VOCAB - _off * _TV)
    _chunk_calls.append(_make_chunk(_off, _nb, _width))
    _off += _nb


def kernel(x, tok_table, pos_table, ln_scale, ln_bias, W_out):
    batch, seq = x.shape
    idx = x.reshape(seq).astype(jnp.int32)
    emb = _make_sc_gather()(idx, tok_table)
    h = _tc_ln(
        emb,
        pos_table[:seq],
        ln_scale.reshape(1, EMB),
        ln_bias.reshape(1, EMB),
    )
    parts = [call(h, W_out) for call in _chunk_calls]
    logits = jnp.concatenate(parts, axis=1)
    return logits.reshape(batch, seq, VOCAB)


# speedup vs baseline: 2.1296x; 1.5432x over previous
import functools

import jax
import jax.numpy as jnp
from jax import lax
from jax.experimental import pallas as pl
from jax.experimental.pallas import tpu as pltpu
from jax.experimental.pallas import tpu_sc as plsc

VOCAB = 50257
EMB = 1024
SEQ = 2048

_NC = 2
_NS = 16
_NW = _NC * _NS
_B_PER_W = SEQ // _NW


@functools.cache
def _make_sc_gather():
    @functools.partial(
        pl.kernel,
        mesh=plsc.VectorSubcoreMesh(
            core_axis_name="c", subcore_axis_name="s", num_cores=_NC
        ),
        out_type=jax.ShapeDtypeStruct((SEQ, EMB), jnp.float32),
        scratch_types=[
            pltpu.VMEM((_B_PER_W,), jnp.int32),
            pltpu.VMEM((_B_PER_W, EMB), jnp.float32),
            pltpu.SemaphoreType.DMA,
        ],
    )
    def _sc_gather(idx_hbm, table_hbm, out_hbm, idx_v, rows_v, sem):
        wid = lax.axis_index("s") * _NC + lax.axis_index("c")
        base = wid * _B_PER_W
        pltpu.sync_copy(idx_hbm.at[pl.ds(base, _B_PER_W)], idx_v)
        pltpu.async_copy(table_hbm.at[idx_v], rows_v, sem).wait()
        pltpu.sync_copy(rows_v, out_hbm.at[pl.ds(base, _B_PER_W)])

    return _sc_gather


_TV = 1024
_NV = (VOCAB + _TV - 1) // _TV


def _ln_body(emb_ref, pos_ref, scale_ref, bias_ref, h_ref):
    h = emb_ref[...] + pos_ref[...]
    mean = jnp.mean(h, axis=1, keepdims=True)
    hc = h - mean
    var = jnp.mean(hc * hc, axis=1, keepdims=True)
    hn = hc * lax.rsqrt(var + 1e-5)
    h_ref[...] = (hn * scale_ref[...] + bias_ref[...]).astype(jnp.bfloat16)


_tc_ln = pl.pallas_call(
    _ln_body,
    out_shape=jax.ShapeDtypeStruct((SEQ, EMB), jnp.bfloat16),
)


def _mm_body(h_ref, w_ref, out_ref):
    out_ref[...] = lax.dot_general(
        h_ref[...],
        w_ref[...].astype(jnp.bfloat16),
        (((1,), (1,)), ((), ())),
        preferred_element_type=jnp.float32,
    )


_tc_mm = pl.pallas_call(
    _mm_body,
    grid=(_NV,),
    in_specs=[
        pl.BlockSpec((SEQ, EMB), lambda i: (0, 0)),
        pl.BlockSpec((_TV, EMB), lambda i: (i, 0)),
    ],
    out_specs=pl.BlockSpec((SEQ, _TV), lambda i: (0, i)),
    out_shape=jax.ShapeDtypeStruct((SEQ, VOCAB), jnp.float32),
    compiler_params=pltpu.CompilerParams(
        dimension_semantics=("arbitrary",),
    ),
)


def kernel(x, tok_table, pos_table, ln_scale, ln_bias, W_out):
    batch, seq = x.shape
    idx = x.reshape(seq).astype(jnp.int32)
    emb = _make_sc_gather()(idx, tok_table)
    h = _tc_ln(
        emb,
        pos_table[:seq],
        ln_scale.reshape(1, EMB),
        ln_bias.reshape(1, EMB),
    )
    logits = _tc_mm(h, W_out)
    return logits.reshape(batch, seq, VOCAB)


# speedup vs baseline: 2.1634x; 1.0159x over previous
import functools

import jax
import jax.numpy as jnp
from jax import lax
from jax.experimental import pallas as pl
from jax.experimental.pallas import tpu as pltpu
from jax.experimental.pallas import tpu_sc as plsc

VOCAB = 50257
EMB = 1024
SEQ = 2048

_NC = 2
_NS = 16
_NW = _NC * _NS
_B_PER_W = SEQ // _NW


@functools.cache
def _make_sc_gather():
    @functools.partial(
        pl.kernel,
        mesh=plsc.VectorSubcoreMesh(
            core_axis_name="c", subcore_axis_name="s", num_cores=_NC
        ),
        out_type=jax.ShapeDtypeStruct((SEQ, EMB), jnp.float32),
        scratch_types=[
            pltpu.VMEM((_B_PER_W,), jnp.int32),
            pltpu.VMEM((_B_PER_W, EMB), jnp.float32),
            pltpu.SemaphoreType.DMA,
        ],
    )
    def _sc_gather(idx_hbm, table_hbm, out_hbm, idx_v, rows_v, sem):
        wid = lax.axis_index("s") * _NC + lax.axis_index("c")
        base = wid * _B_PER_W
        pltpu.sync_copy(idx_hbm.at[pl.ds(base, _B_PER_W)], idx_v)
        pltpu.async_copy(table_hbm.at[idx_v], rows_v, sem).wait()
        pltpu.sync_copy(rows_v, out_hbm.at[pl.ds(base, _B_PER_W)])

    return _sc_gather


_TV = 1536
_NV = (VOCAB + _TV - 1) // _TV


def _ln_body(emb_ref, pos_ref, scale_ref, bias_ref, h_ref):
    h = emb_ref[...] + pos_ref[...]
    mean = jnp.mean(h, axis=1, keepdims=True)
    hc = h - mean
    var = jnp.mean(hc * hc, axis=1, keepdims=True)
    hn = hc * lax.rsqrt(var + 1e-5)
    h_ref[...] = (hn * scale_ref[...] + bias_ref[...]).astype(jnp.bfloat16)


_tc_ln = pl.pallas_call(
    _ln_body,
    out_shape=jax.ShapeDtypeStruct((SEQ, EMB), jnp.bfloat16),
)


def _mm_body(h_ref, w_ref, out_ref):
    out_ref[...] = lax.dot_general(
        h_ref[...],
        w_ref[...].astype(jnp.bfloat16),
        (((1,), (1,)), ((), ())),
        preferred_element_type=jnp.float32,
    )


_tc_mm = pl.pallas_call(
    _mm_body,
    grid=(_NV,),
    in_specs=[
        pl.BlockSpec((SEQ, EMB), lambda i: (0, 0)),
        pl.BlockSpec((_TV, EMB), lambda i: (i, 0)),
    ],
    out_specs=pl.BlockSpec((SEQ, _TV), lambda i: (0, i)),
    out_shape=jax.ShapeDtypeStruct((SEQ, VOCAB), jnp.float32),
    compiler_params=pltpu.CompilerParams(
        dimension_semantics=("arbitrary",),
    ),
)


def kernel(x, tok_table, pos_table, ln_scale, ln_bias, W_out):
    batch, seq = x.shape
    idx = x.reshape(seq).astype(jnp.int32)
    emb = _make_sc_gather()(idx, tok_table)
    h = _tc_ln(
        emb,
        pos_table[:seq],
        ln_scale.reshape(1, EMB),
        ln_bias.reshape(1, EMB),
    )
    logits = _tc_mm(h, W_out)
    return logits.reshape(batch, seq, VOCAB)
